# baseline (speedup 1.0000x reference)
import functools

import jax
import jax.numpy as jnp
from jax import lax
from jax.experimental import pallas as pl
from jax.experimental.pallas import tpu as pltpu
from jax.experimental.pallas import tpu_sc as plsc

_D = 32
_NW = 32
_HB = 2048


@functools.partial(jax.jit, static_argnums=(3, 4))
def _gather_weight(table, idxf, wf, n_b, n_l):
    n_jobs = n_l * (n_b // _HB)
    n_rounds = (n_jobs + _NW - 1) // _NW
    mesh = plsc.VectorSubcoreMesh(core_axis_name="c", subcore_axis_name="s")

    @functools.partial(
        pl.kernel,
        mesh=mesh,
        out_type=jax.ShapeDtypeStruct((n_b, n_l, _D), jnp.float32),
        compiler_params=pltpu.CompilerParams(use_tc_tiling_on_sc=False),
        scratch_types=[
            pltpu.VMEM((_HB,), jnp.int32),
            pltpu.VMEM((_HB,), jnp.float32),
            pltpu.VMEM((_HB, _D), jnp.float32),
            pltpu.SemaphoreType.DMA,
        ],
    )
    def k(table_hbm, idx_hbm, w_hbm, out_hbm, idxf_v, wf_v, rows_v, sem):
        wid = lax.axis_index("s") * 2 + lax.axis_index("c")

        def round_body(r, carry):
            jid = r * _NW + wid

            @pl.when(jid < n_jobs)
            def _():
                lv = jid // (n_b // _HB)
                b0 = lax.rem(jid, n_b // _HB) * _HB
                base = lv * n_b + b0
                pltpu.sync_copy(idx_hbm.at[pl.ds(base, _HB)], idxf_v)
                pltpu.sync_copy(w_hbm.at[pl.ds(base, _HB)], wf_v)
                pltpu.async_copy(table_hbm.at[idxf_v], rows_v, sem).wait()

                def group_body(g16, c):
                    base16 = g16 * 16
                    wvec = wf_v[pl.ds(base16, 16)]
                    for j in range(16):
                        wb = lax.broadcast(wvec[j], (16,))
                        i = base16 + j
                        rows_v[i, 0:16] = rows_v[i, 0:16] * wb
                        rows_v[i, 16:32] = rows_v[i, 16:32] * wb
                    return c

                lax.fori_loop(0, _HB // 16, group_body, 0)
                pltpu.sync_copy(rows_v, out_hbm.at[pl.ds(b0, _HB), lv, :])

            return carry

        lax.fori_loop(0, n_rounds, round_body, 0)

    return k(table, idxf, wf)


def kernel(target_indices, target_weights, embedding_weight):
    b, l = target_indices.shape
    i = jnp.arange(b * l, dtype=jnp.int32)
    lpos = i // b
    bpos = i % b
    idx_t, w_t = jax.lax.optimization_barrier(
        (target_indices.astype(jnp.int32).T, target_weights.T)
    )
    idxf = idx_t[lpos, bpos]
    wf = w_t[lpos, bpos]
    return _gather_weight(embedding_weight, idxf, wf, b, l)


# speedup vs baseline: 1.0003x; 1.0003x over previous
import functools

import jax
import jax.numpy as jnp
from jax import lax
from jax.experimental import pallas as pl
from jax.experimental.pallas import tpu as pltpu
from jax.experimental.pallas import tpu_sc as plsc

_D = 32
_NW = 32
_HB = 2048


@functools.partial(jax.jit, static_argnums=(3, 4))
def _gather_weight(table, idxf, wf, n_b, n_l):
    n_jobs = n_l * (n_b // _HB)
    n_rounds = (n_jobs + _NW - 1) // _NW
    mesh = plsc.VectorSubcoreMesh(core_axis_name="c", subcore_axis_name="s")

    @functools.partial(
        pl.kernel,
        mesh=mesh,
        out_type=jax.ShapeDtypeStruct((n_b, n_l, _D), jnp.float32),
        compiler_params=pltpu.CompilerParams(use_tc_tiling_on_sc=False),
        scratch_types=[
            pltpu.VMEM((_HB,), jnp.int32),
            pltpu.VMEM((_HB,), jnp.float32),
            pltpu.VMEM((_HB, _D), jnp.float32),
            pltpu.SemaphoreType.DMA,
        ],
    )
    def k(table_hbm, idx_hbm, w_hbm, out_hbm, idxf_v, wf_v, rows_v, sem):
        wid = lax.axis_index("s") * 2 + lax.axis_index("c")

        def round_body(r, carry):
            jid = r * _NW + wid

            @pl.when(jid < n_jobs)
            def _():
                lv = jid // (n_b // _HB)
                b0 = lax.rem(jid, n_b // _HB) * _HB
                base = lv * n_b + b0
                pltpu.sync_copy(idx_hbm.at[pl.ds(base, _HB)], idxf_v)
                pltpu.sync_copy(w_hbm.at[pl.ds(base, _HB)], wf_v)
                pltpu.async_copy(table_hbm.at[idxf_v], rows_v, sem).wait()

                def group_body(g16, c):
                    base16 = g16 * 16
                    wvec = wf_v[pl.ds(base16, 16)]
                    for j in range(16):
                        wb = lax.broadcast(wvec[j], (16,))
                        i = base16 + j
                        rows_v[i, 0:16] = rows_v[i, 0:16] * wb
                        rows_v[i, 16:32] = rows_v[i, 16:32] * wb
                    return c

                lax.fori_loop(0, _HB // 16, group_body, 0)
                pltpu.sync_copy(rows_v, out_hbm.at[pl.ds(b0, _HB), lv, :])

            return carry

        lax.fori_loop(0, n_rounds, round_body, 0)

    return k(table, idxf, wf)


def kernel(target_indices, target_weights, embedding_weight):
    b, l = target_indices.shape
    pi = jnp.arange(2 * b * l, dtype=jnp.int32)
    half = pi // 2
    pairs = jnp.where(pi % 2 == 0, half // b, half % b).reshape(b * l, 2)
    idx_t, w_t = jax.lax.optimization_barrier(
        (target_indices.astype(jnp.int32).T, target_weights.T)
    )
    dn = lax.GatherDimensionNumbers(
        offset_dims=(), collapsed_slice_dims=(0, 1), start_index_map=(0, 1)
    )
    idxf = lax.gather(
        idx_t, pairs, dn, (1, 1), mode=lax.GatherScatterMode.PROMISE_IN_BOUNDS
    )
    wf = lax.gather(
        w_t, pairs, dn, (1, 1), mode=lax.GatherScatterMode.PROMISE_IN_BOUNDS
    )
    return _gather_weight(embedding_weight, idxf, wf, b, l)


# speedup vs baseline: 1.0684x; 1.0680x over previous
import functools

import jax
import jax.numpy as jnp
from jax import lax
from jax.experimental import pallas as pl
from jax.experimental.pallas import tpu as pltpu
from jax.experimental.pallas import tpu_sc as plsc

_D = 32
_NW = 32
_HB = 2048


@functools.partial(jax.jit, static_argnums=(2, 3))
def _flatten_lb(idx_t, w_t, n_l, n_b):
    bpw = n_b // _NW
    mesh = plsc.VectorSubcoreMesh(core_axis_name="c", subcore_axis_name="s")

    @functools.partial(
        pl.kernel,
        mesh=mesh,
        out_type=(
            jax.ShapeDtypeStruct((n_l * n_b,), jnp.int32),
            jax.ShapeDtypeStruct((n_l * n_b,), jnp.float32),
        ),
        compiler_params=pltpu.CompilerParams(use_tc_tiling_on_sc=True),
        scratch_types=[
            pltpu.VMEM((n_l, bpw), jnp.int32),
            pltpu.VMEM((n_l, bpw), jnp.float32),
        ],
    )
    def k(idx_hbm, w_hbm, idxf_hbm, wf_hbm, idx_v, w_v):
        wid = lax.axis_index("s") * 2 + lax.axis_index("c")
        b0 = wid * bpw
        pltpu.sync_copy(idx_hbm.at[:, pl.ds(b0, bpw)], idx_v)
        pltpu.sync_copy(w_hbm.at[:, pl.ds(b0, bpw)], w_v)

        def out_body(l, c):
            pltpu.sync_copy(idx_v.at[l], idxf_hbm.at[pl.ds(l * n_b + b0, bpw)])
            pltpu.sync_copy(w_v.at[l], wf_hbm.at[pl.ds(l * n_b + b0, bpw)])
            return c

        lax.fori_loop(0, n_l, out_body, 0)

    return k(idx_t, w_t)


@functools.partial(jax.jit, static_argnums=(3, 4))
def _gather_weight(table, idxf, wf, n_b, n_l):
    n_jobs = n_l * (n_b // _HB)
    n_rounds = (n_jobs + _NW - 1) // _NW
    mesh = plsc.VectorSubcoreMesh(core_axis_name="c", subcore_axis_name="s")

    @functools.partial(
        pl.kernel,
        mesh=mesh,
        out_type=jax.ShapeDtypeStruct((n_b, n_l, _D), jnp.float32),
        compiler_params=pltpu.CompilerParams(use_tc_tiling_on_sc=False),
        scratch_types=[
            pltpu.VMEM((_HB,), jnp.int32),
            pltpu.VMEM((_HB,), jnp.float32),
            pltpu.VMEM((_HB, _D), jnp.float32),
            pltpu.SemaphoreType.DMA,
        ],
    )
    def k(table_hbm, idx_hbm, w_hbm, out_hbm, idxf_v, wf_v, rows_v, sem):
        wid = lax.axis_index("s") * 2 + lax.axis_index("c")

        def round_body(r, carry):
            jid = r * _NW + wid

            @pl.when(jid < n_jobs)
            def _():
                lv = jid // (n_b // _HB)
                b0 = lax.rem(jid, n_b // _HB) * _HB
                base = lv * n_b + b0
                pltpu.sync_copy(idx_hbm.at[pl.ds(base, _HB)], idxf_v)
                pltpu.sync_copy(w_hbm.at[pl.ds(base, _HB)], wf_v)
                pltpu.async_copy(table_hbm.at[idxf_v], rows_v, sem).wait()

                def group_body(g16, c):
                    base16 = g16 * 16
                    wvec = wf_v[pl.ds(base16, 16)]
                    for j in range(16):
                        wb = lax.broadcast(wvec[j], (16,))
                        i = base16 + j
                        rows_v[i, 0:16] = rows_v[i, 0:16] * wb
                        rows_v[i, 16:32] = rows_v[i, 16:32] * wb
                    return c

                lax.fori_loop(0, _HB // 16, group_body, 0)
                pltpu.sync_copy(rows_v, out_hbm.at[pl.ds(b0, _HB), lv, :])

            return carry

        lax.fori_loop(0, n_rounds, round_body, 0)

    return k(table, idxf, wf)


def kernel(target_indices, target_weights, embedding_weight):
    b, l = target_indices.shape
    idx_t, w_t = jax.lax.optimization_barrier(
        (target_indices.astype(jnp.int32).T, target_weights.T)
    )
    idxf, wf = _flatten_lb(idx_t, w_t, l, b)
    return _gather_weight(embedding_weight, idxf, wf, b, l)
